# baseline (speedup 1.0000x reference)
import functools

import jax
import jax.numpy as jnp
from jax import lax
from jax.experimental import pallas as pl
from jax.experimental.pallas import tpu as pltpu
from jax.experimental.pallas import tpu_sc as plsc

_NC = 2
_NS = 16
_NW = _NC * _NS
_CH = 128


@functools.lru_cache(maxsize=None)
def _build(BN, V, D, N):
    rows_per_w = BN // _NW
    n_ch = rows_per_w // _CH
    n_pairs = n_ch // 2
    mesh = plsc.VectorSubcoreMesh(core_axis_name="c", subcore_axis_name="s")

    @functools.partial(
        pl.kernel,
        mesh=mesh,
        out_type=jax.ShapeDtypeStruct((BN, D), jnp.float32),
        scratch_types=[
            pltpu.VMEM((n_ch, _CH), jnp.int32),
            pltpu.VMEM((_CH, D), jnp.float32),
            pltpu.VMEM((_CH, D), jnp.float32),
            pltpu.VMEM_SHARED((N + _CH, D), jnp.float32),
            pltpu.SemaphoreType.DMA,
            pltpu.SemaphoreType.DMA,
            pltpu.SemaphoreType.DMA,
            pltpu.SemaphoreType.DMA,
        ],
        compiler_params=pltpu.CompilerParams(use_tc_tiling_on_sc=False),
    )
    def emb(idx_hbm, tok_hbm, pos_hbm, out_hbm,
            idx_all, rows0, rows1, pos_sh, g0, g1, o0, o1):
        cid = lax.axis_index("c")
        sid = lax.axis_index("s")
        wid = sid * _NC + cid
        base = wid * rows_per_w
        rows = (rows0, rows1)
        gsem = (g0, g1)
        osem = (o0, o1)

        @pl.when(sid == 0)
        def _():
            pltpu.sync_copy(pos_hbm, pos_sh)

        plsc.subcore_barrier()

        pltpu.sync_copy(idx_hbm.at[pl.ds(wid * n_ch, n_ch)], idx_all)

        def prefill_and_gather(c, b):
            o_c = lax.rem(c * _CH, N)
            pltpu.sync_copy(pos_sh.at[pl.ds(o_c, _CH)], rows[b])
            pltpu.async_copy(tok_hbm.at[idx_all.at[c]], rows[b], gsem[b],
                             add=True)

        def wait_gather(c, b):
            pltpu.make_async_copy(tok_hbm.at[idx_all.at[c]], rows[b],
                                  gsem[b]).wait()

        def issue_writeout(c, b):
            pltpu.async_copy(rows[b], out_hbm.at[pl.ds(base + c * _CH, _CH)],
                             osem[b])

        def wait_writeout(b):
            pltpu.make_async_copy(rows[b], out_hbm.at[pl.ds(base, _CH)],
                                  osem[b]).wait()

        prefill_and_gather(0, 0)

        def pair_body(j, carry):
            c = 2 * j
            wait_gather(c, 0)
            issue_writeout(c, 0)

            @pl.when(j > 0)
            def _():
                wait_writeout(1)

            prefill_and_gather(c + 1, 1)

            wait_gather(c + 1, 1)
            issue_writeout(c + 1, 1)
            wait_writeout(0)

            @pl.when(j < n_pairs - 1)
            def _():
                prefill_and_gather(c + 2, 0)

            return carry

        lax.fori_loop(0, n_pairs, pair_body, 0)
        wait_writeout(1)

    return emb


def kernel(indices, tok_emb_table, pos_emb_table):
    B, N = indices.shape
    V, D = tok_emb_table.shape
    BN = B * N
    idx2d = indices.reshape(BN // _CH, _CH).astype(jnp.int32)
    pos = pos_emb_table[:N].astype(jnp.float32)
    pos_ext = jnp.concatenate([pos, pos[:_CH]], axis=0)
    tok_lin = jnp.concatenate(
        [tok_emb_table[0::2], tok_emb_table[1::2]], axis=1).reshape(V, D)
    out = _build(BN, V, D, N)(idx2d, tok_lin, pos_ext)
    return out.reshape(B, N, D)


# speedup vs baseline: 7.7357x; 7.7357x over previous
import functools

import jax
import jax.numpy as jnp
from jax import lax
from jax.experimental import pallas as pl
from jax.experimental.pallas import tpu as pltpu
from jax.experimental.pallas import tpu_sc as plsc

_NC = 2
_NS = 16
_NW = _NC * _NS
_CH = 128


@functools.lru_cache(maxsize=None)
def _build(BN, V, D, N):
    rows_per_w = BN // _NW
    n_ch = rows_per_w // _CH
    _NB = 4
    n_quads = n_ch // _NB
    mesh = plsc.VectorSubcoreMesh(core_axis_name="c", subcore_axis_name="s")

    @functools.partial(
        pl.kernel,
        mesh=mesh,
        out_type=jax.ShapeDtypeStruct((BN, D), jnp.float32),
        scratch_types=[
            pltpu.VMEM((n_ch, _CH), jnp.int32),
            pltpu.VMEM((_CH, D), jnp.float32),
            pltpu.VMEM((_CH, D), jnp.float32),
            pltpu.VMEM((_CH, D), jnp.float32),
            pltpu.VMEM((_CH, D), jnp.float32),
            pltpu.VMEM_SHARED((N + _CH, D), jnp.float32),
            pltpu.SemaphoreType.DMA,
            pltpu.SemaphoreType.DMA,
            pltpu.SemaphoreType.DMA,
            pltpu.SemaphoreType.DMA,
            pltpu.SemaphoreType.DMA,
            pltpu.SemaphoreType.DMA,
            pltpu.SemaphoreType.DMA,
            pltpu.SemaphoreType.DMA,
        ],
        compiler_params=pltpu.CompilerParams(use_tc_tiling_on_sc=False),
    )
    def emb(idx_hbm, tok_hbm, pos_hbm, out_hbm,
            idx_all, rows0, rows1, rows2, rows3, pos_sh,
            g0, g1, g2, g3, o0, o1, o2, o3):
        cid = lax.axis_index("c")
        sid = lax.axis_index("s")
        wid = sid * _NC + cid
        base = wid * rows_per_w
        rows = (rows0, rows1, rows2, rows3)
        gsem = (g0, g1, g2, g3)
        osem = (o0, o1, o2, o3)

        @pl.when(sid == 0)
        def _():
            pltpu.sync_copy(pos_hbm, pos_sh)

        plsc.subcore_barrier()

        pltpu.sync_copy(idx_hbm.at[pl.ds(wid * n_ch, n_ch)], idx_all)

        def prefill_and_gather(c, b):
            o_c = lax.rem(c * _CH, N)
            pltpu.sync_copy(pos_sh.at[pl.ds(o_c, _CH)], rows[b])
            pltpu.async_copy(tok_hbm.at[idx_all.at[c]], rows[b], gsem[b],
                             add=True)

        def wait_gather(c, b):
            pltpu.make_async_copy(tok_hbm.at[idx_all.at[c]], rows[b],
                                  gsem[b]).wait()

        def issue_writeout(c, b):
            pltpu.async_copy(rows[b], out_hbm.at[pl.ds(base + c * _CH, _CH)],
                             osem[b])

        def wait_writeout(b):
            pltpu.make_async_copy(rows[b], out_hbm.at[pl.ds(base, _CH)],
                                  osem[b]).wait()

        prefill_and_gather(0, 0)
        prefill_and_gather(1, 1)
        prefill_and_gather(2, 2)

        def quad_body(j, carry):
            for b in range(_NB):
                c = _NB * j + b
                wait_gather(c, b)
                issue_writeout(c, b)
                nb = (b + 3) % _NB

                @pl.when(c + 3 < n_ch)
                def _():
                    @pl.when(c >= 1)
                    def _():
                        wait_writeout(nb)
                    prefill_and_gather(c + 3, nb)

            return carry

        lax.fori_loop(0, n_quads, quad_body, 0)
        for b in range(_NB):
            wait_writeout(b)

    return emb


def kernel(indices, tok_emb_table, pos_emb_table):
    B, N = indices.shape
    V, D = tok_emb_table.shape
    BN = B * N
    idx2d = indices.reshape(BN // _CH, _CH).astype(jnp.int32)
    pos = pos_emb_table[:N].astype(jnp.float32)
    pos_ext = jnp.concatenate([pos, pos[:_CH]], axis=0)
    out = _build(BN, V, D, N)(idx2d, tok_emb_table, pos_ext)
    return out.reshape(B, N, D)


# speedup vs baseline: 7.7434x; 1.0010x over previous
import functools

import jax
import jax.numpy as jnp
from jax import lax
from jax.experimental import pallas as pl
from jax.experimental.pallas import tpu as pltpu
from jax.experimental.pallas import tpu_sc as plsc

_NC = 2
_NS = 16
_NW = _NC * _NS
_CH = 128


@functools.lru_cache(maxsize=None)
def _build(BN, V, D, N):
    rows_per_w = BN // _NW
    n_ch = rows_per_w // _CH
    _NB = 8
    n_quads = n_ch // _NB
    mesh = plsc.VectorSubcoreMesh(core_axis_name="c", subcore_axis_name="s")

    @functools.partial(
        pl.kernel,
        mesh=mesh,
        out_type=jax.ShapeDtypeStruct((BN, D), jnp.float32),
        scratch_types=[
            pltpu.VMEM((n_ch, _CH), jnp.int32),
            pltpu.VMEM((_CH, D), jnp.float32),
            pltpu.VMEM((_CH, D), jnp.float32),
            pltpu.VMEM((_CH, D), jnp.float32),
            pltpu.VMEM((_CH, D), jnp.float32),
            pltpu.VMEM((_CH, D), jnp.float32),
            pltpu.VMEM((_CH, D), jnp.float32),
            pltpu.VMEM((_CH, D), jnp.float32),
            pltpu.VMEM((_CH, D), jnp.float32),
            pltpu.VMEM_SHARED((N + _CH, D), jnp.float32),
            pltpu.SemaphoreType.DMA,
            pltpu.SemaphoreType.DMA,
            pltpu.SemaphoreType.DMA,
            pltpu.SemaphoreType.DMA,
            pltpu.SemaphoreType.DMA,
            pltpu.SemaphoreType.DMA,
            pltpu.SemaphoreType.DMA,
            pltpu.SemaphoreType.DMA,
            pltpu.SemaphoreType.DMA,
            pltpu.SemaphoreType.DMA,
            pltpu.SemaphoreType.DMA,
            pltpu.SemaphoreType.DMA,
            pltpu.SemaphoreType.DMA,
            pltpu.SemaphoreType.DMA,
            pltpu.SemaphoreType.DMA,
            pltpu.SemaphoreType.DMA,
        ],
        compiler_params=pltpu.CompilerParams(use_tc_tiling_on_sc=False),
    )
    def emb(idx_hbm, tok_hbm, pos_hbm, out_hbm,
            idx_all, rows0, rows1, rows2, rows3, rows4, rows5, rows6, rows7,
            pos_sh, g0, g1, g2, g3, g4, g5, g6, g7,
            o0, o1, o2, o3, o4, o5, o6, o7):
        cid = lax.axis_index("c")
        sid = lax.axis_index("s")
        wid = sid * _NC + cid
        base = wid * rows_per_w
        rows = (rows0, rows1, rows2, rows3, rows4, rows5, rows6, rows7)
        gsem = (g0, g1, g2, g3, g4, g5, g6, g7)
        osem = (o0, o1, o2, o3, o4, o5, o6, o7)

        @pl.when(sid == 0)
        def _():
            pltpu.sync_copy(pos_hbm, pos_sh)

        plsc.subcore_barrier()

        pltpu.sync_copy(idx_hbm.at[pl.ds(wid * n_ch, n_ch)], idx_all)

        def prefill_and_gather(c, b):
            o_c = lax.rem(c * _CH, N)
            pltpu.sync_copy(pos_sh.at[pl.ds(o_c, _CH)], rows[b])
            pltpu.async_copy(tok_hbm.at[idx_all.at[c]], rows[b], gsem[b],
                             add=True)

        def wait_gather(c, b):
            pltpu.make_async_copy(tok_hbm.at[idx_all.at[c]], rows[b],
                                  gsem[b]).wait()

        def issue_writeout(c, b):
            pltpu.async_copy(rows[b], out_hbm.at[pl.ds(base + c * _CH, _CH)],
                             osem[b])

        def wait_writeout(b):
            pltpu.make_async_copy(rows[b], out_hbm.at[pl.ds(base, _CH)],
                                  osem[b]).wait()

        for c0 in range(_NB - 1):
            prefill_and_gather(c0, c0)

        def quad_body(j, carry):
            for b in range(_NB):
                c = _NB * j + b
                wait_gather(c, b)
                issue_writeout(c, b)
                nb = (b + _NB - 1) % _NB

                @pl.when(c + _NB - 1 < n_ch)
                def _():
                    @pl.when(c >= 1)
                    def _():
                        wait_writeout(nb)
                    prefill_and_gather(c + _NB - 1, nb)

            return carry

        lax.fori_loop(0, n_quads, quad_body, 0)
        for b in range(_NB):
            wait_writeout(b)

    return emb


def kernel(indices, tok_emb_table, pos_emb_table):
    B, N = indices.shape
    V, D = tok_emb_table.shape
    BN = B * N
    idx2d = indices.reshape(BN // _CH, _CH).astype(jnp.int32)
    pos = pos_emb_table[:N].astype(jnp.float32)
    pos_ext = jnp.concatenate([pos, pos[:_CH]], axis=0)
    out = _build(BN, V, D, N)(idx2d, tok_emb_table, pos_ext)
    return out.reshape(B, N, D)
